# baseline (speedup 1.0000x reference)
import functools

import jax
import jax.numpy as jnp
from jax import lax
from jax.experimental import pallas as pl
from jax.experimental.pallas import tpu as pltpu
from jax.experimental.pallas import tpu_sc as plsc

B = 4096
LSEQ = 200
VOCAB = 1000
VPAD = 1024
DIM = 128
NOUT = 100

LT = LSEQ // 8
VT = VPAD // 128

NC = 2
NS = 16
LANES = 16
NW = NC * NS
ROWS_PER_W = B // NW
G = LANES
NG = ROWS_PER_W // G


def _sc_histogram(ids3):
    mesh = plsc.VectorSubcoreMesh(
        core_axis_name="c", subcore_axis_name="s",
        num_cores=NC, num_subcores=NS)

    @functools.partial(
        pl.kernel,
        mesh=mesh,
        compiler_params=pltpu.CompilerParams(needs_layout_passes=False),
        out_type=jax.ShapeDtypeStruct((B // 8, VT, 8, 128), jnp.float32),
        scratch_types=[
            pltpu.VMEM((LT, 8, 128), jnp.int32),
            pltpu.VMEM((3, 2, VT, 8, 128), jnp.float32),
            pltpu.SemaphoreType.DMA,
            pltpu.SemaphoreType.DMA,
            pltpu.SemaphoreType.DMA,
            pltpu.SemaphoreType.DMA,
            pltpu.SemaphoreType.DMA,
            pltpu.SemaphoreType.DMA,
            pltpu.SemaphoreType.DMA,
        ],
    )
    def hist_kernel(ids_hbm, zeros_hbm, counts_hbm, slab, cnt, sem_in,
                    sem_o0, sem_o1, sem_o2, sem_z0, sem_z1, sem_z2):
        wid = lax.axis_index("s") * NC + lax.axis_index("c")
        col0 = wid * ROWS_PER_W
        jb0 = wid * (ROWS_PER_W // 8)
        iota = lax.iota(jnp.int32, LANES)
        h_lane = jnp.right_shift(iota, 3)
        r_lane = jnp.bitwise_and(iota, 7)
        ones = jnp.ones((LANES,), jnp.float32)
        sems_o = (sem_o0, sem_o1, sem_o2)
        sems_z = (sem_z0, sem_z1, sem_z2)

        def fetch(i, c):
            pltpu.async_copy(ids_hbm.at[i, :, pl.ds(col0, ROWS_PER_W)],
                             slab.at[i], sem_in)
            return c
        lax.fori_loop(0, LT, fetch, 0)

        for s in range(3):
            pltpu.async_copy(zeros_hbm, cnt.at[s], sems_z[s])

        def drain_in(i, c):
            pltpu.make_async_copy(
                ids_hbm.at[i, :, pl.ds(col0, ROWS_PER_W)],
                slab.at[i], sem_in).wait()
            return c
        lax.fori_loop(0, LT, drain_in, 0)

        def scan_group(cnt_v, gcol):
            def pos(i, c):
                for r in range(8):
                    idv = slab[i, r, pl.ds(gcol, LANES)]
                    t = jnp.right_shift(idv, 7)
                    cc = jnp.bitwise_and(idv, 127)
                    plsc.addupdate_scatter(
                        cnt_v, [h_lane, t, r_lane, cc], ones)
                return c
            lax.fori_loop(0, LT, pos, 0)

        def wait_out(s, g):
            for h in range(2):
                pltpu.make_async_copy(
                    cnt.at[s, h], counts_hbm.at[jb0 + g * 2 + h],
                    sems_o[s]).wait()

        for g in range(NG):
            s = g % 3
            pltpu.make_async_copy(zeros_hbm, cnt.at[s], sems_z[s]).wait()
            scan_group(cnt.at[s], g * G)
            for h in range(2):
                pltpu.async_copy(
                    cnt.at[s, h], counts_hbm.at[jb0 + g * 2 + h],
                    sems_o[s])
            if g >= 1 and g <= NG - 3:
                s2 = (g - 1) % 3
                wait_out(s2, g - 1)
                pltpu.async_copy(zeros_hbm, cnt.at[s2], sems_z[s2])

        for g in range(NG - 3, NG):
            wait_out(g % 3, g)

    return hist_kernel(ids3, jnp.zeros((2, VT, 8, 128), jnp.float32))


def _tc_body(cnt_ref, emb_ref, w_ref, b_ref, out_ref):
    m = jnp.dot(emb_ref[...], w_ref[...],
                preferred_element_type=jnp.float32,
                precision=lax.Precision.HIGHEST)
    blk = cnt_ref.shape[0] * 8
    acc = jnp.zeros((blk, NOUT), jnp.float32)
    for t in range(VT):
        lhs = cnt_ref[:, t, :, :].reshape(blk, 128)
        acc = acc + jnp.dot(lhs, m[t * 128:(t + 1) * 128, :],
                            preferred_element_type=jnp.float32)
    out_ref[...] = acc * (1.0 / LSEQ) + b_ref[...]


def _tc_logits(counts4, emb_pad, w, b2d):
    grid = 8
    jblk = B // 8 // grid
    return pl.pallas_call(
        _tc_body,
        grid=(grid,),
        in_specs=[
            pl.BlockSpec((jblk, VT, 8, 128), lambda i: (i, 0, 0, 0)),
            pl.BlockSpec((VPAD, DIM), lambda i: (0, 0)),
            pl.BlockSpec((DIM, NOUT), lambda i: (0, 0)),
            pl.BlockSpec((1, NOUT), lambda i: (0, 0)),
        ],
        out_specs=pl.BlockSpec((jblk * 8, NOUT), lambda i: (i, 0)),
        out_shape=jax.ShapeDtypeStruct((B, NOUT), jnp.float32),
    )(counts4, emb_pad, w, b2d)


def kernel(input_ids, embedding_table, W, b):
    ids = input_ids.astype(jnp.int32)
    ids3 = ids.T.reshape(LT, 8, B)
    counts4 = _sc_histogram(ids3)
    emb_pad = jnp.pad(embedding_table, ((0, VPAD - VOCAB), (0, 0)))
    return _tc_logits(counts4, emb_pad, W, b.reshape(1, NOUT))


# speedup vs baseline: 1.2747x; 1.2747x over previous
import functools

import jax
import jax.numpy as jnp
from jax import lax
from jax.experimental import pallas as pl
from jax.experimental.pallas import tpu as pltpu
from jax.experimental.pallas import tpu_sc as plsc

B = 4096
LSEQ = 200
VOCAB = 1000
VPAD = 1024
DIM = 128
NOUT = 100

LT = LSEQ // 8
VT = VPAD // 128

NC = 2
NS = 16
LANES = 16
NW = NC * NS
ROWS_PER_W = B // NW
G = LANES
NG = ROWS_PER_W // G


def _sc_histogram(ids3):
    mesh = plsc.VectorSubcoreMesh(
        core_axis_name="c", subcore_axis_name="s",
        num_cores=NC, num_subcores=NS)

    @functools.partial(
        pl.kernel,
        mesh=mesh,
        compiler_params=pltpu.CompilerParams(needs_layout_passes=False),
        out_type=jax.ShapeDtypeStruct((B // 8, VT, 8, 128), jnp.float32),
        scratch_types=[
            pltpu.VMEM((LT, 8, 128), jnp.int32),
            pltpu.VMEM((2, VT, 8, 128), jnp.float32),
            pltpu.VMEM((2, VT, 8, 128), jnp.float32),
            pltpu.SemaphoreType.DMA,
            pltpu.SemaphoreType.DMA,
            pltpu.SemaphoreType.DMA,
        ],
    )
    def hist_kernel(ids_hbm, counts_hbm, slab, cnt0, cnt1,
                    sem_in, sem_o0, sem_o1):
        wid = lax.axis_index("s") * NC + lax.axis_index("c")
        col0 = wid * ROWS_PER_W
        jb0 = wid * (ROWS_PER_W // 8)
        iota = lax.iota(jnp.int32, LANES)
        h_lane = jnp.right_shift(iota, 3)
        r_lane = jnp.bitwise_and(iota, 7)
        ones = jnp.ones((LANES,), jnp.float32)
        zeros = jnp.zeros((LANES,), jnp.float32)

        def fetch(i, c):
            pltpu.async_copy(ids_hbm.at[i, :, pl.ds(col0, ROWS_PER_W)],
                             slab.at[i], sem_in)
            return c
        lax.fori_loop(0, LT, fetch, 0)

        def zero_all(j, c):
            for u in range(8):
                q = j * 8 + u
                cnt0[q >> 9, (q >> 6) & 7, (q >> 3) & 7,
                     pl.ds((q & 7) * LANES, LANES)] = zeros
                cnt1[q >> 9, (q >> 6) & 7, (q >> 3) & 7,
                     pl.ds((q & 7) * LANES, LANES)] = zeros
            return c
        lax.fori_loop(0, 2 * VT * 8 * 8 // 8, zero_all, 0)

        def drain_in(i, c):
            pltpu.make_async_copy(
                ids_hbm.at[i, :, pl.ds(col0, ROWS_PER_W)],
                slab.at[i], sem_in).wait()
            return c
        lax.fori_loop(0, LT, drain_in, 0)

        cnts = (cnt0, cnt1)
        sems = (sem_o0, sem_o1)

        def scan_group(cnt_v, gcol, accumulate):
            def pos(i, c):
                for r in range(8):
                    idv = slab[i, r, pl.ds(gcol, LANES)]
                    t = jnp.right_shift(idv, 7)
                    cc = jnp.bitwise_and(idv, 127)
                    idx = [h_lane, t, r_lane, cc]
                    if accumulate:
                        plsc.addupdate_scatter(cnt_v, idx, ones)
                    else:
                        plsc.store_scatter(cnt_v, idx, zeros)
                return c
            lax.fori_loop(0, LT, pos, 0)

        def drain_out(cnt_v, sem_o, g):
            for h in range(2):
                pltpu.make_async_copy(
                    cnt_v.at[h], counts_hbm.at[jb0 + g * 2 + h],
                    sem_o).wait()

        for g in range(NG):
            cnt_v, sem_o = cnts[g % 2], sems[g % 2]
            if g >= 2:
                drain_out(cnt_v, sem_o, g - 2)
                scan_group(cnt_v, (g - 2) * G, accumulate=False)
            scan_group(cnt_v, g * G, accumulate=True)
            for h in range(2):
                pltpu.async_copy(
                    cnt_v.at[h], counts_hbm.at[jb0 + g * 2 + h], sem_o)

        drain_out(cnts[(NG - 2) % 2], sems[(NG - 2) % 2], NG - 2)
        drain_out(cnts[(NG - 1) % 2], sems[(NG - 1) % 2], NG - 1)

    return hist_kernel(ids3)


def _premul_body(emb_ref, w_ref, m_ref):
    m_ref[...] = jnp.dot(emb_ref[...], w_ref[...],
                         preferred_element_type=jnp.float32,
                         precision=lax.Precision.HIGHEST)


def _tc_premul(emb_pad, w):
    return pl.pallas_call(
        _premul_body,
        out_shape=jax.ShapeDtypeStruct((VPAD, NOUT), jnp.float32),
    )(emb_pad, w)


def _tc_body(cnt_ref, m_ref, b_ref, out_ref):
    blk = cnt_ref.shape[0] * 8
    acc = jnp.zeros((blk, NOUT), jnp.float32)
    for t in range(VT):
        lhs = cnt_ref[:, t, :, :].reshape(blk, 128)
        acc = acc + jnp.dot(lhs, m_ref[t * 128:(t + 1) * 128, :],
                            preferred_element_type=jnp.float32)
    out_ref[...] = acc * (1.0 / LSEQ) + b_ref[...]


def _tc_logits(counts4, m, b2d):
    grid = 8
    jblk = B // 8 // grid
    return pl.pallas_call(
        _tc_body,
        grid=(grid,),
        in_specs=[
            pl.BlockSpec((jblk, VT, 8, 128), lambda i: (i, 0, 0, 0)),
            pl.BlockSpec((VPAD, NOUT), lambda i: (0, 0)),
            pl.BlockSpec((1, NOUT), lambda i: (0, 0)),
        ],
        out_specs=pl.BlockSpec((jblk * 8, NOUT), lambda i: (i, 0)),
        out_shape=jax.ShapeDtypeStruct((B, NOUT), jnp.float32),
    )(counts4, m, b2d)


def kernel(input_ids, embedding_table, W, b):
    ids = input_ids.astype(jnp.int32)
    ids3 = ids.T.reshape(LT, 8, B)
    counts4 = _sc_histogram(ids3)
    emb_pad = jnp.pad(embedding_table, ((0, VPAD - VOCAB), (0, 0)))
    m = _tc_premul(emb_pad, W)
    return _tc_logits(counts4, m, b.reshape(1, NOUT))


# speedup vs baseline: 1.2904x; 1.0123x over previous
import functools

import jax
import jax.numpy as jnp
from jax import lax
from jax.experimental import pallas as pl
from jax.experimental.pallas import tpu as pltpu
from jax.experimental.pallas import tpu_sc as plsc

B = 4096
LSEQ = 200
VOCAB = 1000
VPAD = 1024
DIM = 128
NOUT = 100

LT = LSEQ // 8
VT = VPAD // 128

NC = 2
NS = 16
LANES = 16
NW = NC * NS
ROWS_PER_W = B // NW
G = LANES
NG = ROWS_PER_W // G


def _sc_histogram(ids3, goff, bh):
    rows_w = bh // NW
    ng = rows_w // G
    mesh = plsc.VectorSubcoreMesh(
        core_axis_name="c", subcore_axis_name="s",
        num_cores=NC, num_subcores=NS)

    @functools.partial(
        pl.kernel,
        mesh=mesh,
        compiler_params=pltpu.CompilerParams(needs_layout_passes=False),
        out_type=jax.ShapeDtypeStruct((bh // 8, VT, 8, 128), jnp.float32),
        scratch_types=[
            pltpu.VMEM((LT, 8, 128), jnp.int32),
            pltpu.VMEM((2, VT, 8, 128), jnp.float32),
            pltpu.VMEM((2, VT, 8, 128), jnp.float32),
            pltpu.SemaphoreType.DMA,
            pltpu.SemaphoreType.DMA,
            pltpu.SemaphoreType.DMA,
        ],
    )
    def hist_kernel(ids_hbm, counts_hbm, slab, cnt0, cnt1,
                    sem_in, sem_o0, sem_o1):
        wid = lax.axis_index("s") * NC + lax.axis_index("c")
        col0 = wid * ROWS_PER_W
        jb0 = wid * (rows_w // 8)
        iota = lax.iota(jnp.int32, LANES)
        h_lane = jnp.right_shift(iota, 3)
        r_lane = jnp.bitwise_and(iota, 7)
        ones = jnp.ones((LANES,), jnp.float32)
        zeros = jnp.zeros((LANES,), jnp.float32)

        def fetch(i, c):
            pltpu.async_copy(ids_hbm.at[i, :, pl.ds(col0, ROWS_PER_W)],
                             slab.at[i], sem_in)
            return c
        lax.fori_loop(0, LT, fetch, 0)

        def zero_all(j, c):
            for u in range(8):
                q = j * 8 + u
                cnt0[q >> 9, (q >> 6) & 7, (q >> 3) & 7,
                     pl.ds((q & 7) * LANES, LANES)] = zeros
                cnt1[q >> 9, (q >> 6) & 7, (q >> 3) & 7,
                     pl.ds((q & 7) * LANES, LANES)] = zeros
            return c
        lax.fori_loop(0, 2 * VT * 8 * 8 // 8, zero_all, 0)

        def drain_in(i, c):
            pltpu.make_async_copy(
                ids_hbm.at[i, :, pl.ds(col0, ROWS_PER_W)],
                slab.at[i], sem_in).wait()
            return c
        lax.fori_loop(0, LT, drain_in, 0)

        cnts = (cnt0, cnt1)
        sems = (sem_o0, sem_o1)

        def scan_group(cnt_v, gcol, accumulate):
            def pos(i, c):
                for r in range(8):
                    idv = slab[i, r, pl.ds(gcol, LANES)]
                    t = jnp.right_shift(idv, 7)
                    cc = jnp.bitwise_and(idv, 127)
                    idx = [h_lane, t, r_lane, cc]
                    if accumulate:
                        plsc.addupdate_scatter(cnt_v, idx, ones)
                    else:
                        plsc.store_scatter(cnt_v, idx, zeros)
                return c
            lax.fori_loop(0, LT, pos, 0)

        def drain_out(cnt_v, sem_o, g):
            for h in range(2):
                pltpu.make_async_copy(
                    cnt_v.at[h], counts_hbm.at[jb0 + g * 2 + h],
                    sem_o).wait()

        for g in range(ng):
            cnt_v, sem_o = cnts[g % 2], sems[g % 2]
            if g >= 2:
                drain_out(cnt_v, sem_o, g - 2)
                scan_group(cnt_v, (goff + g - 2) * G, accumulate=False)
            scan_group(cnt_v, (goff + g) * G, accumulate=True)
            for h in range(2):
                pltpu.async_copy(
                    cnt_v.at[h], counts_hbm.at[jb0 + g * 2 + h], sem_o)

        drain_out(cnts[(ng - 2) % 2], sems[(ng - 2) % 2], ng - 2)
        drain_out(cnts[(ng - 1) % 2], sems[(ng - 1) % 2], ng - 1)

    return hist_kernel(ids3)


def _premul_body(emb_ref, w_ref, m_ref):
    m_ref[...] = jnp.dot(emb_ref[...], w_ref[...],
                         preferred_element_type=jnp.float32,
                         precision=lax.Precision.HIGHEST)


def _tc_premul(emb_pad, w):
    return pl.pallas_call(
        _premul_body,
        out_shape=jax.ShapeDtypeStruct((VPAD, NOUT), jnp.float32),
    )(emb_pad, w)


def _tc_body(cnt_ref, m_ref, b_ref, out_ref):
    blk = cnt_ref.shape[0] * 8
    acc = jnp.zeros((blk, NOUT), jnp.float32)
    for t in range(VT):
        lhs = cnt_ref[:, t, :, :].reshape(blk, 128)
        acc = acc + jnp.dot(lhs, m_ref[t * 128:(t + 1) * 128, :],
                            preferred_element_type=jnp.float32)
    out_ref[...] = acc * (1.0 / LSEQ) + b_ref[...]


def _tc_logits(counts4, m, b2d):
    bh = counts4.shape[0] * 8
    jblk = 64
    grid = bh // (jblk * 8)
    return pl.pallas_call(
        _tc_body,
        grid=(grid,),
        in_specs=[
            pl.BlockSpec((jblk, VT, 8, 128), lambda i: (i, 0, 0, 0)),
            pl.BlockSpec((VPAD, NOUT), lambda i: (0, 0)),
            pl.BlockSpec((1, NOUT), lambda i: (0, 0)),
        ],
        out_specs=pl.BlockSpec((jblk * 8, NOUT), lambda i: (i, 0)),
        out_shape=jax.ShapeDtypeStruct((bh, NOUT), jnp.float32),
    )(counts4, m, b2d)


def kernel(input_ids, embedding_table, W, b):
    ids = input_ids.astype(jnp.int32)
    ids3 = ids.T.reshape(LT, 8, B)
    emb_pad = jnp.pad(embedding_table, ((0, VPAD - VOCAB), (0, 0)))
    m = _tc_premul(emb_pad, W)
    b2d = b.reshape(1, NOUT)
    bh = B // 2
    c0 = _sc_histogram(ids3, 0, bh)
    c1 = _sc_histogram(ids3, NG // 2, bh)
    o0 = _tc_logits(c0, m, b2d).reshape(NW, bh // NW, NOUT)
    o1 = _tc_logits(c1, m, b2d).reshape(NW, bh // NW, NOUT)
    return jnp.concatenate([o0, o1], axis=1).reshape(B, NOUT)
